# baseline (speedup 1.0000x reference)
import functools

import jax
import jax.numpy as jnp
from jax import lax
from jax.experimental import pallas as pl
from jax.experimental.pallas import tpu as pltpu
from jax.experimental.pallas import tpu_sc as plsc


def _sortable(bits):
    flip = jnp.full(bits.shape, 0x7FFFFFFF, jnp.int32)
    return jnp.where(bits >= 0, bits, jnp.bitwise_xor(bits, flip))


def _sc_median_build(rows, n, k):
    nw = 32
    rpw = rows // nw
    mesh = plsc.VectorSubcoreMesh(core_axis_name="c", subcore_axis_name="s")
    cap = n + 16
    nreg = 7

    @functools.partial(
        pl.kernel,
        mesh=mesh,
        compiler_params=pltpu.CompilerParams(needs_layout_passes=False),
        out_type=jax.ShapeDtypeStruct((rows,), jnp.int32),
        scratch_types=[
            pltpu.VMEM((cap,), jnp.int32),
            pltpu.VMEM((cap,), jnp.int32),
            pltpu.VMEM((nreg * 4096,), jnp.int32),
            pltpu.VMEM((rpw,), jnp.int32),
        ],
    )
    def sc_kernel(x_hbm, out_hbm, row_v, cand_v, hist_v, res_v):
        wid = lax.axis_index("s") * 2 + lax.axis_index("c")
        lane = lax.iota(jnp.int32, 16)
        ones = jnp.ones((16,), jnp.int32)
        zeros16 = jnp.zeros((16,), jnp.int32)

        def rst(b, _):
            hist_v[pl.ds(b * 16, 16)] = zeros16
            return 0
        lax.fori_loop(0, nreg * 256, rst, 0)

        def find(r):
            def fbody(i, carry):
                acc, t, above = carry
                b = 255 - i
                cb = jnp.sum(hist_v[pl.ds(b * 16, 16)])
                hist_v[pl.ds(b * 16, 16)] = zeros16
                newacc = acc + cb
                hit = (acc < r) & (newacc >= r)
                return (newacc,
                        jnp.where(hit, b, t),
                        jnp.where(hit, acc, above))
            _, t, above = lax.fori_loop(
                0, 256, fbody,
                (jnp.int32(0), jnp.int32(0), jnp.int32(0)))
            return t, above

        def level0(r):
            def hbody(i, _):
                v = row_v[pl.ds(i * 16, 16)]
                s = _sortable(v)
                bn = (s >> 24) + 128
                plsc.addupdate_scatter(hist_v, [bn * 16 + lane], ones)
                return 0
            lax.fori_loop(0, n // 16, hbody, 0)

            def rbody(q, _):
                hs = [hist_v[pl.ds(g * 4096 + q * 16, 16)]
                      for g in range(nreg)]
                acc = ((hs[0] + hs[1]) + (hs[2] + hs[3])
                       + ((hs[4] + hs[5]) + hs[6]))
                for g in range(1, nreg):
                    hist_v[pl.ds(g * 4096 + q * 16, 16)] = zeros16
                hist_v[pl.ds(q * 16, 16)] = acc
                return 0
            lax.fori_loop(0, 256, rbody, 0)

            t, above = find(r)

            def cbody(i, off):
                ss, sels = [], []
                for u in range(8):
                    v = row_v[pl.ds((i * 8 + u) * 16, 16)]
                    s = _sortable(v)
                    ss.append(s)
                    sels.append(((s >> 24) + 128) == t)
                pcs = [jnp.sum(sel.astype(jnp.int32)) for sel in sels]
                for u in range(8):
                    plsc.store_compressed(
                        cand_v.at[pl.ds(off, 16)], ss[u], mask=sels[u])
                    off = off + pcs[u]
                return off
            m2 = lax.fori_loop(0, n // 128, cbody, jnp.int32(0))
            return m2, r - above, t

        def level(src_ref, dst_ref, m, r, lvl):
            shift = 24 - 8 * lvl
            nv = (m + 15) // 16

            def keybin(i):
                s = src_ref[pl.ds(i * 16, 16)]
                return s, (s >> shift) & 0xFF

            def hbody(i, _):
                _, bn = keybin(i)
                plsc.addupdate_scatter(
                    hist_v, [bn * 16 + lane], ones,
                    mask=(i * 16 + lane) < m)
                return 0
            lax.fori_loop(0, nv, hbody, 0)

            t, above = find(r)

            if lvl < 3:
                def cbody(i, off):
                    s, bn = keybin(i)
                    sel = (bn == t) & ((i * 16 + lane) < m)
                    plsc.store_compressed(
                        dst_ref.at[pl.ds(off, 16)], s, mask=sel)
                    return off + jnp.sum(sel.astype(jnp.int32))
                m2 = lax.fori_loop(0, nv, cbody, jnp.int32(0))
            else:
                m2 = m
            return m2, r - above, t

        def emit_result(j, s_ans):
            plsc.store_scatter(
                res_v, [jnp.broadcast_to(j, (16,))],
                jnp.broadcast_to(s_ans, (16,)), mask=(lane == 0))

        def do_row(j, _):
            row = wid * rpw + j
            pltpu.sync_copy(x_hbm.at[row], row_v.at[pl.ds(0, n)])
            m1, r1, t0 = level0(jnp.int32(k))

            @pl.when(m1 <= 16)
            def _():
                keys = cand_v[pl.ds(0, 16)]
                pad = jnp.full((16,), -(2 ** 31), jnp.int32)
                keys = jnp.where(lane < m1, keys, pad)
                sk, _ = plsc.sort_key_val(keys, keys, descending=True)
                s_ans = jnp.sum(jnp.where(lane == r1 - 1, sk, 0))
                emit_result(j, s_ans)

            @pl.when(m1 > 16)
            def _():
                m2, r2, t1 = level(cand_v, row_v, m1, r1, 1)
                m3, r3, t2 = level(row_v, cand_v, m2, r2, 2)
                _, _, t3 = level(cand_v, row_v, m3, r3, 3)
                s_ans = ((t0 - 128) << 24) | (t1 << 16) | (t2 << 8) | t3
                emit_result(j, s_ans)
            return 0
        lax.fori_loop(0, rpw, do_row, 0)
        pltpu.sync_copy(res_v, out_hbm.at[pl.ds(wid * rpw, rpw)])

    return sc_kernel


def _dense_body(s_ref, w_ref, b_ref, o_ref):
    med = lax.bitcast_convert_type(_sortable(s_ref[...]), jnp.float32)
    o_ref[...] = (
        jnp.dot(med, w_ref[...], preferred_element_type=jnp.float32)
        + b_ref[...]
    )


def kernel(x, W, b):
    B, C, H, Wsp = x.shape
    n = H * Wsp
    k = n // 2
    rows = B * C
    xi = lax.bitcast_convert_type(x.reshape(rows, n), jnp.int32)

    s_med = _sc_median_build(rows, n, k)(xi)

    out = pl.pallas_call(
        _dense_body,
        out_shape=jax.ShapeDtypeStruct((B, W.shape[0]), jnp.float32),
    )(s_med.reshape(B, C), W.T, b.reshape(1, -1))
    return out


# speedup vs baseline: 1.5739x; 1.5739x over previous
import functools

import jax
import jax.numpy as jnp
from jax import lax
from jax.experimental import pallas as pl
from jax.experimental.pallas import tpu as pltpu
from jax.experimental.pallas import tpu_sc as plsc


def _sortable(bits):
    flip = jnp.full(bits.shape, 0x7FFFFFFF, jnp.int32)
    return jnp.where(bits >= 0, bits, jnp.bitwise_xor(bits, flip))


def _sc_median_build(rows, n, k):
    nw = 32
    rpw = rows // nw
    mesh = plsc.VectorSubcoreMesh(core_axis_name="c", subcore_axis_name="s")
    cap = n + 16
    nreg = 7

    @functools.partial(
        pl.kernel,
        mesh=mesh,
        compiler_params=pltpu.CompilerParams(needs_layout_passes=False),
        out_type=jax.ShapeDtypeStruct((rows,), jnp.int32),
        scratch_types=[
            pltpu.VMEM((cap,), jnp.int32),
            pltpu.VMEM((cap,), jnp.int32),
            pltpu.VMEM((nreg * 4096,), jnp.int32),
            pltpu.VMEM((rpw,), jnp.int32),
        ],
    )
    def sc_kernel(x_hbm, out_hbm, row_v, cand_v, hist_v, res_v):
        wid = lax.axis_index("s") * 2 + lax.axis_index("c")
        lane = lax.iota(jnp.int32, 16)
        ones = jnp.ones((16,), jnp.int32)
        zeros16 = jnp.zeros((16,), jnp.int32)

        def rst(b, _):
            hist_v[pl.ds(b * 16, 16)] = zeros16
            return 0
        lax.fori_loop(0, nreg * 256, rst, 0)

        def find(r):
            def fbody(i, carry):
                acc, t, above = carry
                b = 255 - i
                cb = jnp.sum(hist_v[pl.ds(b * 16, 16)])
                hist_v[pl.ds(b * 16, 16)] = zeros16
                newacc = acc + cb
                hit = (acc < r) & (newacc >= r)
                return (newacc,
                        jnp.where(hit, b, t),
                        jnp.where(hit, acc, above))
            _, t, above = lax.fori_loop(
                0, 256, fbody,
                (jnp.int32(0), jnp.int32(0), jnp.int32(0)))
            return t, above

        def level0(r):
            def hbody(i, _):
                bns = []
                for u in range(8):
                    v = row_v[pl.ds((i * 8 + u) * 16, 16)]
                    s = _sortable(v)
                    bns.append((s >> 24) + 128)
                for u in range(8):
                    plsc.addupdate_scatter(
                        hist_v, [(u % nreg) * 4096 + bns[u] * 16 + lane],
                        ones)
                return 0
            lax.fori_loop(0, n // 128, hbody, 0)

            def rbody(q, _):
                hs = [hist_v[pl.ds(g * 4096 + q * 16, 16)]
                      for g in range(nreg)]
                acc = ((hs[0] + hs[1]) + (hs[2] + hs[3])
                       + ((hs[4] + hs[5]) + hs[6]))
                for g in range(1, nreg):
                    hist_v[pl.ds(g * 4096 + q * 16, 16)] = zeros16
                hist_v[pl.ds(q * 16, 16)] = acc
                return 0
            lax.fori_loop(0, 256, rbody, 0)

            t, above = find(r)

            def cbody(i, off):
                ss, sels = [], []
                for u in range(8):
                    v = row_v[pl.ds((i * 8 + u) * 16, 16)]
                    s = _sortable(v)
                    ss.append(s)
                    sels.append(((s >> 24) + 128) == t)
                pcs = [jnp.sum(sel.astype(jnp.int32)) for sel in sels]
                for u in range(8):
                    plsc.store_compressed(
                        cand_v.at[pl.ds(off, 16)], ss[u], mask=sels[u])
                    off = off + pcs[u]
                return off
            m2 = lax.fori_loop(0, n // 128, cbody, jnp.int32(0))
            return m2, r - above, t

        def level(src_ref, dst_ref, m, r, lvl):
            shift = 24 - 8 * lvl
            nv = (m + 15) // 16

            def keybin(i):
                s = src_ref[pl.ds(i * 16, 16)]
                return s, (s >> shift) & 0xFF

            def hbody(i, _):
                _, bn = keybin(i)
                plsc.addupdate_scatter(
                    hist_v, [bn * 16 + lane], ones,
                    mask=(i * 16 + lane) < m)
                return 0
            lax.fori_loop(0, nv, hbody, 0)

            t, above = find(r)

            if lvl < 3:
                def cbody(i, off):
                    s, bn = keybin(i)
                    sel = (bn == t) & ((i * 16 + lane) < m)
                    plsc.store_compressed(
                        dst_ref.at[pl.ds(off, 16)], s, mask=sel)
                    return off + jnp.sum(sel.astype(jnp.int32))
                m2 = lax.fori_loop(0, nv, cbody, jnp.int32(0))
            else:
                m2 = m
            return m2, r - above, t

        def emit_result(j, s_ans):
            plsc.store_scatter(
                res_v, [jnp.broadcast_to(j, (16,))],
                jnp.broadcast_to(s_ans, (16,)), mask=(lane == 0))

        def do_row(j, _):
            row = wid * rpw + j
            pltpu.sync_copy(x_hbm.at[row], row_v.at[pl.ds(0, n)])
            m1, r1, t0 = level0(jnp.int32(k))

            @pl.when(m1 <= 16)
            def _():
                keys = cand_v[pl.ds(0, 16)]
                pad = jnp.full((16,), -(2 ** 31), jnp.int32)
                keys = jnp.where(lane < m1, keys, pad)
                sk, _ = plsc.sort_key_val(keys, keys, descending=True)
                s_ans = jnp.sum(jnp.where(lane == r1 - 1, sk, 0))
                emit_result(j, s_ans)

            @pl.when(m1 > 16)
            def _():
                m2, r2, t1 = level(cand_v, row_v, m1, r1, 1)
                m3, r3, t2 = level(row_v, cand_v, m2, r2, 2)
                _, _, t3 = level(cand_v, row_v, m3, r3, 3)
                s_ans = ((t0 - 128) << 24) | (t1 << 16) | (t2 << 8) | t3
                emit_result(j, s_ans)
            return 0
        lax.fori_loop(0, rpw, do_row, 0)
        pltpu.sync_copy(res_v, out_hbm.at[pl.ds(wid * rpw, rpw)])

    return sc_kernel


def _dense_body(s_ref, w_ref, b_ref, o_ref):
    med = lax.bitcast_convert_type(_sortable(s_ref[...]), jnp.float32)
    o_ref[...] = (
        jnp.dot(med, w_ref[...], preferred_element_type=jnp.float32)
        + b_ref[...]
    )


def kernel(x, W, b):
    B, C, H, Wsp = x.shape
    n = H * Wsp
    k = n // 2
    rows = B * C
    xi = lax.bitcast_convert_type(x.reshape(rows, n), jnp.int32)

    s_med = _sc_median_build(rows, n, k)(xi)

    out = pl.pallas_call(
        _dense_body,
        out_shape=jax.ShapeDtypeStruct((B, W.shape[0]), jnp.float32),
    )(s_med.reshape(B, C), W.T, b.reshape(1, -1))
    return out


# speedup vs baseline: 1.5739x; 1.0000x over previous
import functools

import jax
import jax.numpy as jnp
from jax import lax
from jax.experimental import pallas as pl
from jax.experimental.pallas import tpu as pltpu
from jax.experimental.pallas import tpu_sc as plsc


def _sortable(bits):
    flip = jnp.full(bits.shape, 0x7FFFFFFF, jnp.int32)
    return jnp.where(bits >= 0, bits, jnp.bitwise_xor(bits, flip))


def _sc_median_build(rows, n, k):
    nw = 32
    rpw = rows // nw
    mesh = plsc.VectorSubcoreMesh(core_axis_name="c", subcore_axis_name="s")
    cap = n + 16
    nreg = 7

    @functools.partial(
        pl.kernel,
        mesh=mesh,
        compiler_params=pltpu.CompilerParams(needs_layout_passes=False),
        out_type=jax.ShapeDtypeStruct((rows,), jnp.int32),
        scratch_types=[
            pltpu.VMEM((cap,), jnp.int32),
            pltpu.VMEM((cap,), jnp.int32),
            pltpu.VMEM((nreg * 4096,), jnp.int32),
            pltpu.VMEM((rpw,), jnp.int32),
            pltpu.SemaphoreType.DMA,
            pltpu.SemaphoreType.DMA,
        ],
    )
    def sc_kernel(x_hbm, out_hbm, row_v, cand_v, hist_v, res_v,
                  sem1, sem2):
        wid = lax.axis_index("s") * 2 + lax.axis_index("c")
        lane = lax.iota(jnp.int32, 16)
        ones = jnp.ones((16,), jnp.int32)
        zeros16 = jnp.zeros((16,), jnp.int32)

        def rst(b, _):
            hist_v[pl.ds(b * 16, 16)] = zeros16
            return 0
        lax.fori_loop(0, nreg * 256, rst, 0)

        def find(r):
            def fbody(i, carry):
                acc, t, above = carry
                b = 255 - i
                cb = jnp.sum(hist_v[pl.ds(b * 16, 16)])
                hist_v[pl.ds(b * 16, 16)] = zeros16
                newacc = acc + cb
                hit = (acc < r) & (newacc >= r)
                return (newacc,
                        jnp.where(hit, b, t),
                        jnp.where(hit, acc, above))
            _, t, above = lax.fori_loop(
                0, 256, fbody,
                (jnp.int32(0), jnp.int32(0), jnp.int32(0)))
            return t, above

        def hist0_range(lo_blk, hi_blk):
            def hbody(i, _):
                bns = []
                for u in range(8):
                    v = row_v[pl.ds((i * 8 + u) * 16, 16)]
                    s = _sortable(v)
                    bns.append((s >> 24) + 128)
                for u in range(8):
                    plsc.addupdate_scatter(
                        hist_v, [(u % nreg) * 4096 + bns[u] * 16 + lane],
                        ones)
                return 0
            lax.fori_loop(lo_blk, hi_blk, hbody, 0)

        def level0(r):
            def rbody(q, _):
                hs = [hist_v[pl.ds(g * 4096 + q * 16, 16)]
                      for g in range(nreg)]
                acc = ((hs[0] + hs[1]) + (hs[2] + hs[3])
                       + ((hs[4] + hs[5]) + hs[6]))
                for g in range(1, nreg):
                    hist_v[pl.ds(g * 4096 + q * 16, 16)] = zeros16
                hist_v[pl.ds(q * 16, 16)] = acc
                return 0
            lax.fori_loop(0, 256, rbody, 0)

            t, above = find(r)

            def cbody(i, off):
                ss, sels = [], []
                for u in range(8):
                    v = row_v[pl.ds((i * 8 + u) * 16, 16)]
                    s = _sortable(v)
                    ss.append(s)
                    sels.append(((s >> 24) + 128) == t)
                pcs = [jnp.sum(sel.astype(jnp.int32)) for sel in sels]
                for u in range(8):
                    plsc.store_compressed(
                        cand_v.at[pl.ds(off, 16)], ss[u], mask=sels[u])
                    off = off + pcs[u]
                return off
            m2 = lax.fori_loop(0, n // 128, cbody, jnp.int32(0))
            return m2, r - above, t

        def level(src_ref, dst_ref, m, r, lvl):
            shift = 24 - 8 * lvl
            nv = (m + 15) // 16

            def keybin(i):
                s = src_ref[pl.ds(i * 16, 16)]
                return s, (s >> shift) & 0xFF

            def hbody(i, _):
                _, bn = keybin(i)
                plsc.addupdate_scatter(
                    hist_v, [bn * 16 + lane], ones,
                    mask=(i * 16 + lane) < m)
                return 0
            lax.fori_loop(0, nv, hbody, 0)

            t, above = find(r)

            if lvl < 3:
                def cbody(i, off):
                    s, bn = keybin(i)
                    sel = (bn == t) & ((i * 16 + lane) < m)
                    plsc.store_compressed(
                        dst_ref.at[pl.ds(off, 16)], s, mask=sel)
                    return off + jnp.sum(sel.astype(jnp.int32))
                m2 = lax.fori_loop(0, nv, cbody, jnp.int32(0))
            else:
                m2 = m
            return m2, r - above, t

        def emit_result(j, s_ans):
            plsc.store_scatter(
                res_v, [jnp.broadcast_to(j, (16,))],
                jnp.broadcast_to(s_ans, (16,)), mask=(lane == 0))

        def do_row(j, _):
            row = wid * rpw + j
            half = n // 2
            c1 = pltpu.async_copy(
                x_hbm.at[row, pl.ds(0, half)],
                row_v.at[pl.ds(0, half)], sem1)
            c2 = pltpu.async_copy(
                x_hbm.at[row, pl.ds(half, half)],
                row_v.at[pl.ds(half, half)], sem2)
            c1.wait()
            hist0_range(0, n // 256)
            c2.wait()
            hist0_range(n // 256, n // 128)
            m1, r1, t0 = level0(jnp.int32(k))

            @pl.when(m1 <= 16)
            def _():
                keys = cand_v[pl.ds(0, 16)]
                pad = jnp.full((16,), -(2 ** 31), jnp.int32)
                keys = jnp.where(lane < m1, keys, pad)
                sk, _ = plsc.sort_key_val(keys, keys, descending=True)
                s_ans = jnp.sum(jnp.where(lane == r1 - 1, sk, 0))
                emit_result(j, s_ans)

            @pl.when(m1 > 16)
            def _():
                m2, r2, t1 = level(cand_v, row_v, m1, r1, 1)
                m3, r3, t2 = level(row_v, cand_v, m2, r2, 2)
                _, _, t3 = level(cand_v, row_v, m3, r3, 3)
                s_ans = ((t0 - 128) << 24) | (t1 << 16) | (t2 << 8) | t3
                emit_result(j, s_ans)
            return 0
        lax.fori_loop(0, rpw, do_row, 0)
        pltpu.sync_copy(res_v, out_hbm.at[pl.ds(wid * rpw, rpw)])

    return sc_kernel


def _dense_body(s_ref, w_ref, b_ref, o_ref):
    med = lax.bitcast_convert_type(_sortable(s_ref[...]), jnp.float32)
    o_ref[...] = (
        jnp.dot(med, w_ref[...], preferred_element_type=jnp.float32)
        + b_ref[...]
    )


def kernel(x, W, b):
    B, C, H, Wsp = x.shape
    n = H * Wsp
    k = n // 2
    rows = B * C
    xi = lax.bitcast_convert_type(x.reshape(rows, n), jnp.int32)

    s_med = _sc_median_build(rows, n, k)(xi)

    out = pl.pallas_call(
        _dense_body,
        out_shape=jax.ShapeDtypeStruct((B, W.shape[0]), jnp.float32),
    )(s_med.reshape(B, C), W.T, b.reshape(1, -1))
    return out
